# Initial kernel scaffold; baseline (speedup 1.0000x reference)
import jax
import jax.numpy as jnp
from jax.experimental import pallas as pl


def kernel(x, edge_index, W1, b1, W2, b2, W3, b3, W4, b4):
    raise NotImplementedError("write your pallas kernel here")


# speedup vs baseline: 6.9891x; 6.9891x over previous
import functools

import jax
import jax.numpy as jnp
from jax import lax
from jax.experimental import pallas as pl
from jax.experimental.pallas import tpu as pltpu
from jax.experimental.pallas import tpu_sc as plsc

N_PAD = 10240
D = 128
NC = 2
NS = 16
NW = NC * NS
C = 128
CW = 16
ROWS_PER_TILE = N_PAD // NS
BLK = 1024


def _mesh():
    return plsc.VectorSubcoreMesh(core_axis_name="c", subcore_axis_name="s")


def _fill_rows(buf, nrows, width, value16):
    def body(j, carry):
        for i in range(width // 16):
            buf[j, pl.ds(i * 16, 16)] = value16
        return carry
    lax.fori_loop(0, nrows, body, 0)


def _count_call(K):
    @functools.partial(
        pl.kernel,
        out_type=jax.ShapeDtypeStruct((NC, N_PAD, CW), jnp.float32),
        mesh=_mesh(),
        scratch_types=[
            pltpu.VMEM((K, C), jnp.int32),
            pltpu.VMEM((C, CW), jnp.float32),
            pltpu.VMEM((C, CW), jnp.float32),
            pltpu.VMEM_SHARED((N_PAD, CW), jnp.float32),
        ],
    )
    def count(col_hbm, cnt_hbm, col_v, ones_v, zbuf, acc):
        cid = lax.axis_index("c")
        sid = lax.axis_index("s")
        wid = cid * NS + sid
        _fill_rows(ones_v, C, CW, jnp.full((16,), 1.0, jnp.float32))
        _fill_rows(zbuf, C, CW, jnp.zeros((16,), jnp.float32))
        base = sid * ROWS_PER_TILE
        for q in range(ROWS_PER_TILE // C):
            pltpu.sync_copy(zbuf, acc.at[pl.ds(base + q * C, C)])
        plsc.subcore_barrier()
        pltpu.sync_copy(col_hbm.at[wid], col_v)

        def body(j, carry):
            pltpu.sync_copy(ones_v, acc.at[col_v.at[j]], add=True)
            return carry
        lax.fori_loop(0, K, body, 0)
        plsc.subcore_barrier()
        for q in range(ROWS_PER_TILE // C):
            pltpu.sync_copy(acc.at[pl.ds(base + q * C, C)], zbuf)
            pltpu.sync_copy(zbuf, cnt_hbm.at[cid, pl.ds(base + q * C, C)])

    return count


def _agg_call(K):
    assert K % 2 == 0

    @functools.partial(
        pl.kernel,
        out_type=jax.ShapeDtypeStruct((NC, N_PAD, D), jnp.float32),
        mesh=_mesh(),
        scratch_types=[
            pltpu.VMEM((K, C), jnp.int32),
            pltpu.VMEM((2, C), jnp.int32),
            pltpu.VMEM((2, C), jnp.int32),
            pltpu.VMEM((2, C, D), jnp.float32),
            pltpu.VMEM_SHARED((N_PAD, D), jnp.float32),
        ],
    )
    def agg(g_hbm, pk_hbm, p_hbm, pk_v, rowc, colc, gbuf, acc):
        cid = lax.axis_index("c")
        sid = lax.axis_index("s")
        wid = cid * NS + sid
        _fill_rows(gbuf.at[0], C, D, jnp.zeros((16,), jnp.float32))
        base = sid * ROWS_PER_TILE
        for q in range(ROWS_PER_TILE // C):
            pltpu.sync_copy(gbuf.at[0], acc.at[pl.ds(base + q * C, C)])
        plsc.subcore_barrier()
        pltpu.sync_copy(pk_hbm.at[wid], pk_v)

        mask = jnp.full((16,), 0xFFFF, jnp.int32)

        def unpack(j, b):
            for i in range(C // 16):
                v = pk_v[j, pl.ds(i * 16, 16)]
                rowc[b, pl.ds(i * 16, 16)] = lax.bitwise_and(v, mask)
                colc[b, pl.ds(i * 16, 16)] = lax.shift_right_logical(
                    v, jnp.full((16,), 16, jnp.int32))

        _fill_rows(gbuf.at[1], C, D, jnp.zeros((16,), jnp.float32))
        dummy16 = jnp.full((16,), N_PAD - 1, jnp.int32)
        for i in range(C // 16):
            colc[1, pl.ds(i * 16, 16)] = dummy16

        def run(sem0, sem1):
            sems = (sem0, sem1)
            unpack(0, 0)
            pltpu.sync_copy(g_hbm.at[rowc.at[0]], gbuf.at[0])
            pltpu.async_copy(gbuf.at[1], acc.at[colc.at[1]], sem1, add=True)

            def body(i, carry):
                j = i * 2
                for b in range(2):
                    pltpu.async_copy(gbuf.at[b], acc.at[colc.at[b]],
                                     sems[b], add=True)
                    pltpu.make_async_copy(
                        gbuf.at[1 - b], acc.at[colc.at[1 - b]], sems[1 - b]
                    ).wait()
                    jj = jnp.minimum(j + 1 + b, K - 1)
                    unpack(jj, 1 - b)
                    pltpu.sync_copy(g_hbm.at[rowc.at[1 - b]], gbuf.at[1 - b])
                return carry
            lax.fori_loop(0, K // 2, body, 0)
            pltpu.make_async_copy(
                gbuf.at[1], acc.at[colc.at[1]], sem1).wait()

        pl.run_scoped(run, pltpu.SemaphoreType.DMA, pltpu.SemaphoreType.DMA)

        plsc.subcore_barrier()
        for q in range(ROWS_PER_TILE // C):
            pltpu.sync_copy(acc.at[pl.ds(base + q * C, C)], gbuf.at[0])
            pltpu.sync_copy(gbuf.at[0], p_hbm.at[cid, pl.ds(base + q * C, C)])

    return agg


def _dis_of(cnt_ref):
    return lax.rsqrt(cnt_ref[0, :, 0:1] + cnt_ref[1, :, 0:1] + 1.0)


def _mm_first(x, W, cnt):
    def body(x_ref, w_ref, cnt_ref, o_ref):
        dis = _dis_of(cnt_ref)
        o_ref[...] = dis * jnp.dot(x_ref[...], w_ref[...],
                                   preferred_element_type=jnp.float32)
    return pl.pallas_call(
        body,
        grid=(N_PAD // BLK,),
        in_specs=[
            pl.BlockSpec((BLK, D), lambda i: (i, 0)),
            pl.BlockSpec((D, D), lambda i: (0, 0)),
            pl.BlockSpec((NC, BLK, CW), lambda i: (0, i, 0)),
        ],
        out_specs=pl.BlockSpec((BLK, D), lambda i: (i, 0)),
        out_shape=jax.ShapeDtypeStruct((N_PAD, D), jnp.float32),
    )(x, W, cnt)


def _mm_mid(p, g, b, cnt, W):
    def body(p_ref, g_ref, b_ref, cnt_ref, w_ref, o_ref):
        dis = _dis_of(cnt_ref)
        h = dis * (p_ref[0] + p_ref[1] + g_ref[...]) + b_ref[...]
        a = jnp.maximum(h, 0.0)
        o_ref[...] = dis * jnp.dot(a, w_ref[...],
                                   preferred_element_type=jnp.float32)
    return pl.pallas_call(
        body,
        grid=(N_PAD // BLK,),
        in_specs=[
            pl.BlockSpec((NC, BLK, D), lambda i: (0, i, 0)),
            pl.BlockSpec((BLK, D), lambda i: (i, 0)),
            pl.BlockSpec((1, D), lambda i: (0, 0)),
            pl.BlockSpec((NC, BLK, CW), lambda i: (0, i, 0)),
            pl.BlockSpec((D, D), lambda i: (0, 0)),
        ],
        out_specs=pl.BlockSpec((BLK, D), lambda i: (i, 0)),
        out_shape=jax.ShapeDtypeStruct((N_PAD, D), jnp.float32),
    )(p, g, b, cnt, W)


def _mm_last(p, g, b, cnt):
    def body(p_ref, g_ref, b_ref, cnt_ref, o_ref):
        dis = _dis_of(cnt_ref)
        o_ref[...] = dis * (p_ref[0] + p_ref[1] + g_ref[...]) + b_ref[...]
    return pl.pallas_call(
        body,
        grid=(N_PAD // BLK,),
        in_specs=[
            pl.BlockSpec((NC, BLK, D), lambda i: (0, i, 0)),
            pl.BlockSpec((BLK, D), lambda i: (i, 0)),
            pl.BlockSpec((1, D), lambda i: (0, 0)),
            pl.BlockSpec((NC, BLK, CW), lambda i: (0, i, 0)),
        ],
        out_specs=pl.BlockSpec((BLK, D), lambda i: (i, 0)),
        out_shape=jax.ShapeDtypeStruct((N_PAD, D), jnp.float32),
    )(p, g, b, cnt)


def kernel(x, edge_index, W1, b1, W2, b2, W3, b3, W4, b4):
    n = x.shape[0]
    e = edge_index.shape[1]
    x_pad = jnp.pad(x, ((0, N_PAD - n), (0, 0)))

    slab = NW * C * 2
    e_pad = -(-e // slab) * slab
    pad = jnp.full((e_pad - e,), n, jnp.int32)
    rowp = jnp.concatenate([edge_index[0].astype(jnp.int32), pad])
    colp = jnp.concatenate([edge_index[1].astype(jnp.int32), pad])
    K = e_pad // (NW * C)
    pk = (rowp | (colp << 16)).reshape(NW, K, C)
    colp3 = colp.reshape(NW, K, C)

    count = _count_call(K)
    agg = _agg_call(K)

    cnt = count(colp3)
    b1r, b2r, b3r, b4r = (b.reshape(1, D) for b in (b1, b2, b3, b4))

    g = _mm_first(x_pad, W1, cnt)
    p = agg(g, pk)
    g = _mm_mid(p, g, b1r, cnt, W2)
    p = agg(g, pk)
    g = _mm_mid(p, g, b2r, cnt, W3)
    p = agg(g, pk)
    g = _mm_mid(p, g, b3r, cnt, W4)
    p = agg(g, pk)
    out = _mm_last(p, g, b4r, cnt)
    return out[:n]
